# baseline (speedup 1.0000x reference)
import functools

import jax
import jax.numpy as jnp
from jax import lax
from jax.experimental import pallas as pl
from jax.experimental.pallas import tpu as pltpu
from jax.experimental.pallas import tpu_sc as plsc

_IDXW = 128


def _make_gather(B, C):
    info = plsc.get_sparse_core_info()
    _NC, _NS, _L = info.num_cores, info.num_subcores, info.num_lanes
    _NW = _NC * _NS
    b_per_w = B // _NW
    n_chunks = b_per_w // _IDXW
    mesh = plsc.VectorSubcoreMesh(core_axis_name="c", subcore_axis_name="s")

    @functools.partial(
        pl.kernel,
        out_type=jax.ShapeDtypeStruct((B,), jnp.float32),
        mesh=mesh,
        scratch_types=[
            pltpu.VMEM((b_per_w,), jnp.int32),
            pltpu.VMEM((n_chunks, _IDXW), jnp.int32),
            pltpu.VMEM((b_per_w,), jnp.float32),
            pltpu.SemaphoreType.DMA,
        ],
    )
    def gather_k(lower_flat, targets_hbm, g_hbm, tgt_v, idx_v, g_v, sem):
        wid = lax.axis_index("s") * _NC + lax.axis_index("c")
        base = wid * b_per_w
        pltpu.sync_copy(targets_hbm.at[pl.ds(base, b_per_w)], tgt_v)
        iota = lax.iota(jnp.int32, _L)
        for i in range(b_per_w // _L):
            t = tgt_v[pl.ds(i * _L, _L)]
            rows = (base + i * _L) + iota
            j, off = divmod(i * _L, _IDXW)
            idx_v[j, pl.ds(off, _L)] = rows * C + t
        cps = [
            pltpu.async_copy(
                lower_flat.at[idx_v.at[j]],
                g_v.at[pl.ds(j * _IDXW, _IDXW)],
                sem,
            )
            for j in range(n_chunks)
        ]
        for cp in cps:
            cp.wait()
        pltpu.sync_copy(g_v, g_hbm.at[pl.ds(base, b_per_w)])

    return gather_k


def _make_dense(B, C, RB, interpret=False):
    def body(u_ref, g_ref, t_ref, o_ref):
        col = lax.broadcasted_iota(jnp.int32, (RB, C), 1)
        res = u_ref[...] - g_ref[...]
        o_ref[...] = jnp.where(col == t_ref[...], 0.0, res)

    return pl.pallas_call(
        body,
        grid=(B // RB,),
        in_specs=[
            pl.BlockSpec((RB, C), lambda i: (i, 0)),
            pl.BlockSpec((RB, 1), lambda i: (i, 0)),
            pl.BlockSpec((RB, 1), lambda i: (i, 0)),
        ],
        out_specs=pl.BlockSpec((RB, C), lambda i: (i, 0)),
        out_shape=jax.ShapeDtypeStruct((B, C), jnp.float32),
        interpret=interpret,
    )


def kernel(x, lower, upper, targets):
    B, C = upper.shape
    g = _make_gather(B, C)(lower.reshape(-1), targets)
    res = _make_dense(B, C, 512)(upper, g.reshape(B, 1), targets.reshape(B, 1))
    return (x, res)


# speedup vs baseline: 2.2889x; 2.2889x over previous
import functools

import jax
import jax.numpy as jnp
from jax import lax
from jax.experimental import pallas as pl
from jax.experimental.pallas import tpu as pltpu
from jax.experimental.pallas import tpu_sc as plsc

_IDXW = 128


def _make_gather(B, C):
    info = plsc.get_sparse_core_info()
    _NC, _NS, _L = info.num_cores, info.num_subcores, info.num_lanes
    _NW = _NC * _NS
    b_per_w = B // _NW
    n_chunks = b_per_w // _IDXW
    mesh = plsc.VectorSubcoreMesh(core_axis_name="c", subcore_axis_name="s")

    @functools.partial(
        pl.kernel,
        out_type=jax.ShapeDtypeStruct((B,), jnp.float32),
        mesh=mesh,
        scratch_types=[
            pltpu.VMEM((b_per_w,), jnp.int32),
            pltpu.VMEM((n_chunks, _IDXW), jnp.int32),
            pltpu.VMEM((b_per_w,), jnp.float32),
            pltpu.SemaphoreType.DMA,
        ],
    )
    def gather_k(lowerT_flat, targets_hbm, g_hbm, tgt_v, idx_v, g_v, sem):
        wid = lax.axis_index("s") * _NC + lax.axis_index("c")
        base = wid * b_per_w
        pltpu.sync_copy(targets_hbm.at[pl.ds(base, b_per_w)], tgt_v)
        iota = lax.iota(jnp.int32, _L)
        for i in range(b_per_w // _L):
            t = tgt_v[pl.ds(i * _L, _L)]
            b_ids = (base + i * _L) + iota
            j, off = divmod(i * _L, _IDXW)
            idx_v[j, pl.ds(off, _L)] = t * B + b_ids
        cps = [
            pltpu.async_copy(
                lowerT_flat.at[idx_v.at[j]],
                g_v.at[pl.ds(j * _IDXW, _IDXW)],
                sem,
            )
            for j in range(n_chunks)
        ]
        for cp in cps:
            cp.wait()
        pltpu.sync_copy(g_v, g_hbm.at[pl.ds(base, b_per_w)])

    return gather_k


def _make_dense_T(B, C, BB, interpret=False):
    def body(u_ref, g_ref, t_ref, o_ref):
        row = lax.broadcasted_iota(jnp.int32, (C, BB), 0)
        res = u_ref[...] - g_ref[...]
        o_ref[...] = jnp.where(row == t_ref[...], 0.0, res)

    return pl.pallas_call(
        body,
        grid=(B // BB,),
        in_specs=[
            pl.BlockSpec((C, BB), lambda j: (0, j)),
            pl.BlockSpec((BB,), lambda j: (j,)),
            pl.BlockSpec((BB,), lambda j: (j,)),
        ],
        out_specs=pl.BlockSpec((C, BB), lambda j: (0, j)),
        out_shape=jax.ShapeDtypeStruct((C, B), jnp.float32),
        interpret=interpret,
    )


def kernel(x, lower, upper, targets):
    B, C = upper.shape
    lowerT_flat = lower.T.reshape(-1)
    g = _make_gather(B, C)(lowerT_flat, targets)
    resT = _make_dense_T(B, C, 512)(upper.T, g, targets)
    return (x, resT.T)


# speedup vs baseline: 3.3233x; 1.4519x over previous
import functools

import jax
import jax.numpy as jnp
from jax import lax
from jax.experimental import pallas as pl
from jax.experimental.pallas import tpu as pltpu
from jax.experimental.pallas import tpu_sc as plsc

_IDXW = 128


def _make_gather(B, C):
    info = plsc.get_sparse_core_info()
    _NC, _NS, _L = info.num_cores, info.num_subcores, info.num_lanes
    _NW = _NC * _NS
    b_per_w = B // _NW
    n_chunks = b_per_w // _IDXW
    mesh = plsc.VectorSubcoreMesh(core_axis_name="c", subcore_axis_name="s")

    @functools.partial(
        pl.kernel,
        out_type=jax.ShapeDtypeStruct((B,), jnp.float32),
        mesh=mesh,
        scratch_types=[
            pltpu.VMEM((b_per_w,), jnp.int32),
            pltpu.VMEM((n_chunks, _IDXW), jnp.int32),
            pltpu.VMEM((b_per_w,), jnp.float32),
            pltpu.SemaphoreType.DMA,
        ],
    )
    def gather_k(lower_raw, targets_hbm, g_hbm, tgt_v, idx_v, g_v, sem):
        wid = lax.axis_index("s") * _NC + lax.axis_index("c")
        base = wid * b_per_w
        pltpu.sync_copy(targets_hbm.at[pl.ds(base, b_per_w)], tgt_v)
        iota = lax.iota(jnp.int32, _L)
        row_stride = 8 * (B // 128) * 128
        for i in range(b_per_w // _L):
            t = tgt_v[pl.ds(i * _L, _L)]
            b_ids = (base + i * _L) + iota
            j, off = divmod(i * _L, _IDXW)
            idx_v[j, pl.ds(off, _L)] = (
                (t >> 3) * row_stride
                + (b_ids >> 7) * 1024
                + (t & 7) * 128
                + (b_ids & 127)
            )
        cps = [
            pltpu.async_copy(
                lower_raw.at[idx_v.at[j]],
                g_v.at[pl.ds(j * _IDXW, _IDXW)],
                sem,
            )
            for j in range(n_chunks)
        ]
        for cp in cps:
            cp.wait()
        pltpu.sync_copy(g_v, g_hbm.at[pl.ds(base, b_per_w)])

    return gather_k


def _make_dense_T(B, C, BB, interpret=False):
    def body(u_ref, g_ref, t_ref, o_ref):
        row = lax.broadcasted_iota(jnp.int32, (C, BB), 0)
        res = u_ref[...] - g_ref[...]
        o_ref[...] = jnp.where(row == t_ref[...], 0.0, res)

    return pl.pallas_call(
        body,
        grid=(B // BB,),
        in_specs=[
            pl.BlockSpec((C, BB), lambda j: (0, j)),
            pl.BlockSpec((BB,), lambda j: (j,)),
            pl.BlockSpec((BB,), lambda j: (j,)),
        ],
        out_specs=pl.BlockSpec((C, BB), lambda j: (0, j)),
        out_shape=jax.ShapeDtypeStruct((C, B), jnp.float32),
        interpret=interpret,
    )


def kernel(x, lower, upper, targets):
    B, C = upper.shape
    lower_raw = (
        lower.T.reshape(C // 8, 8, B // 128, 128)
        .transpose(0, 2, 1, 3)
        .reshape(-1)
    )
    g = _make_gather(B, C)(lower_raw, targets)
    resT = _make_dense_T(B, C, 512)(upper.T, g, targets)
    return (x, resT.T)


# speedup vs baseline: 3.4237x; 1.0302x over previous
import functools

import jax
import jax.numpy as jnp
from jax import lax
from jax.experimental import pallas as pl
from jax.experimental.pallas import tpu as pltpu
from jax.experimental.pallas import tpu_sc as plsc

_IDXW = 128


def _make_gather(B, C):
    info = plsc.get_sparse_core_info()
    _NC, _NS, _L = info.num_cores, info.num_subcores, info.num_lanes
    _NW = _NC * _NS
    b_per_w = B // _NW
    n_chunks = b_per_w // _IDXW
    mesh = plsc.VectorSubcoreMesh(core_axis_name="c", subcore_axis_name="s")

    @functools.partial(
        pl.kernel,
        out_type=jax.ShapeDtypeStruct((B,), jnp.float32),
        mesh=mesh,
        scratch_types=[
            pltpu.VMEM((b_per_w,), jnp.int32),
            pltpu.VMEM((n_chunks, _IDXW), jnp.int32),
            pltpu.VMEM((b_per_w,), jnp.float32),
            pltpu.SemaphoreType.DMA,
        ],
    )
    def gather_k(lower_raw, targets_hbm, g_hbm, tgt_v, idx_v, g_v, sem):
        wid = lax.axis_index("s") * _NC + lax.axis_index("c")
        base = wid * b_per_w
        pltpu.sync_copy(targets_hbm.at[pl.ds(base, b_per_w)], tgt_v)
        iota = lax.iota(jnp.int32, _L)
        row_stride = 8 * (B // 128) * 128
        for i in range(b_per_w // _L):
            t = tgt_v[pl.ds(i * _L, _L)]
            b_ids = (base + i * _L) + iota
            j, off = divmod(i * _L, _IDXW)
            idx_v[j, pl.ds(off, _L)] = (
                (t >> 3) * row_stride
                + (b_ids >> 7) * 1024
                + (t & 7) * 128
                + (b_ids & 127)
            )
        cps = [
            pltpu.async_copy(
                lower_raw.at[idx_v.at[j]],
                g_v.at[pl.ds(j * _IDXW, _IDXW)],
                sem,
            )
            for j in range(n_chunks)
        ]
        for cp in cps:
            cp.wait()
        pltpu.sync_copy(g_v, g_hbm.at[pl.ds(base, b_per_w)])

    return gather_k


def _make_xcopy(n):
    info = plsc.get_sparse_core_info()
    _NW = info.num_cores * info.num_subcores
    per_w = n // _NW
    n_chunks = 8
    chunk = per_w // n_chunks
    mesh = plsc.VectorSubcoreMesh(core_axis_name="c", subcore_axis_name="s")

    @functools.partial(
        pl.kernel,
        out_type=jax.ShapeDtypeStruct((n,), jnp.float32),
        mesh=mesh,
        scratch_types=[
            pltpu.VMEM((chunk,), jnp.float32),
            pltpu.VMEM((chunk,), jnp.float32),
            pltpu.SemaphoreType.DMA,
            pltpu.SemaphoreType.DMA,
            pltpu.SemaphoreType.DMA,
            pltpu.SemaphoreType.DMA,
        ],
    )
    def xcopy_k(src, dst, buf0, buf1, sr0, sr1, sw0, sw1):
        wid = lax.axis_index("s") * info.num_cores + lax.axis_index("c")
        base = wid * per_w
        bufs, srs, sws = (buf0, buf1), (sr0, sr1), (sw0, sw1)
        rd, wr = {}, {}
        for k in range(n_chunks):
            p = k % 2
            if k >= 2:
                wr[k - 2].wait()
            rd[k] = pltpu.async_copy(
                src.at[pl.ds(base + k * chunk, chunk)], bufs[p], srs[p]
            )
            if k >= 1:
                rd[k - 1].wait()
                q = (k - 1) % 2
                wr[k - 1] = pltpu.async_copy(
                    bufs[q], dst.at[pl.ds(base + (k - 1) * chunk, chunk)], sws[q]
                )
        k = n_chunks - 1
        rd[k].wait()
        wr[k] = pltpu.async_copy(
            bufs[k % 2], dst.at[pl.ds(base + k * chunk, chunk)], sws[k % 2]
        )
        wr[k - 1].wait()
        wr[k].wait()

    return xcopy_k


def _make_dense_T(B, C, BB, interpret=False):
    def body(u_ref, g_ref, t_ref, o_ref):
        row = lax.broadcasted_iota(jnp.int32, (C, BB), 0)
        res = u_ref[...] - g_ref[...]
        o_ref[...] = jnp.where(row == t_ref[...], 0.0, res)

    return pl.pallas_call(
        body,
        grid=(B // BB,),
        in_specs=[
            pl.BlockSpec((C, BB), lambda j: (0, j)),
            pl.BlockSpec((BB,), lambda j: (j,)),
            pl.BlockSpec((BB,), lambda j: (j,)),
        ],
        out_specs=pl.BlockSpec((C, BB), lambda j: (0, j)),
        out_shape=jax.ShapeDtypeStruct((C, B), jnp.float32),
        interpret=interpret,
    )


def kernel(x, lower, upper, targets):
    B, C = upper.shape
    lower_raw = (
        lower.T.reshape(C // 8, 8, B // 128, 128)
        .transpose(0, 2, 1, 3)
        .reshape(-1)
    )
    g = _make_gather(B, C)(lower_raw, targets)
    x_raw = (
        x.T.reshape(C // 8, 8, B // 128, 128)
        .transpose(0, 2, 1, 3)
        .reshape(-1)
    )
    x_flat = _make_xcopy(B * C)(x_raw)
    x_out = (
        x_flat.reshape(C // 8, B // 128, 8, 128)
        .transpose(0, 2, 1, 3)
        .reshape(C, B)
        .T
    )
    resT = _make_dense_T(B, C, 512)(upper.T, g, targets)
    return (x_out, resT.T)
